# baseline (speedup 1.0000x reference)
import functools

import numpy as np
import jax
import jax.numpy as jnp
from jax import lax
from jax.experimental import pallas as pl
from jax.experimental.pallas import tpu as pltpu
from jax.experimental.pallas import tpu_sc as plsc

N = 128
G3 = N * N * N
HALF = G3 // 2
DUMMY = HALF
EPS = 1e-6
SIGMA = 2

NPTS = 100000
NTILES = 16
PT = 6272
NPAD = NTILES * PT
GRP = PT // 16
CH_GRP = 14
NCHUNK = GRP // CH_GRP

_k = np.arange(N)
_ang = 2.0 * np.pi * ((np.outer(_k, _k)) % N) / N
_C = np.cos(_ang)
_S = np.sin(_ang)
_f = np.fft.fftfreq(N, d=1.0 / N)

C_M = np.asarray(_C, dtype=np.float32)
S_M = np.asarray(_S, dtype=np.float32)
CZS = np.asarray(_C * _f[None, :], dtype=np.float32)
SZS = np.asarray(_S * _f[None, :], dtype=np.float32)
CYS = np.asarray(_C * _f[:, None], dtype=np.float32)
SYS = np.asarray(_S * _f[:, None], dtype=np.float32)
CXS = CYS
SXS = SYS
FX2 = np.asarray((_f ** 2)[:, None], dtype=np.float32)
FYZ2 = np.asarray(((_f ** 2)[:, None] + (_f ** 2)[None, :]).reshape(1, -1),
                  dtype=np.float32)

@functools.cache
def _sc_mesh():
    return plsc.VectorSubcoreMesh(core_axis_name="c", subcore_axis_name="s",
                                  num_cores=2, num_subcores=16)


def _corner_vectors(c0x, c0y, c0z):
    ax0 = c0x * (N * N)
    ax1 = ((c0x + 1) & (N - 1)) * (N * N)
    by0 = c0y * N
    by1 = ((c0y + 1) & (N - 1)) * N
    cz1 = (c0z + 1) & (N - 1)
    e = (ax0 + by0, ax0 + by1, ax1 + by0, ax1 + by1)
    return e, c0z, cz1


def _prep_points(pts_hbm, cid, base_pt, fb, cb):
    for ax in range(3):
        off = (cid * 3 + ax) * NPAD + base_pt
        pltpu.sync_copy(pts_hbm.at[pl.ds(off, PT)], fb[ax])

    def prep(i, _):
        po = i * 16
        for ax in range(3):
            t = fb[ax][pl.ds(po, 16)] * float(N)
            c0 = t.astype(jnp.int32)
            cb[ax][pl.ds(po, 16)] = c0
            fb[ax][pl.ds(po, 16)] = t - c0.astype(jnp.float32)
        return 0

    lax.fori_loop(0, GRP, prep, 0)


@functools.cache
def _sc_scatter_kernel():
    return pl.kernel(
        _sc_scatter_body,
        out_type=jax.ShapeDtypeStruct((2 * 3 * G3,), jnp.float32),
        mesh=_sc_mesh(),
        scratch_types=[
            pltpu.VMEM_SHARED((HALF + 8,), jnp.float32),
            [pltpu.VMEM((PT,), jnp.float32) for _ in range(3)],
            [pltpu.VMEM((PT,), jnp.int32) for _ in range(3)],
            pltpu.VMEM((PT,), jnp.float32),
            [pltpu.VMEM((CH_GRP, 128), jnp.int32) for _ in range(4)],
            [pltpu.VMEM((CH_GRP, 128), jnp.float32) for _ in range(4)],
            pltpu.VMEM((4096,), jnp.float32),
            pltpu.SemaphoreType.DMA,
            pltpu.SemaphoreType.DMA,
        ],
    )


def _sc_scatter_body(pts_hbm, nrm_hbm, ras_hbm, grid_sh, fb, cb, nch, idxbs,
                     valbs, zb, sem, sem2):
    cid = lax.axis_index("c")
    sid = lax.axis_index("s")
    base_pt = sid * PT
    z16 = jnp.zeros((16,), jnp.float32)

    def zfill(i, _):
        zb[pl.ds(i * 16, 16)] = z16
        return 0

    lax.fori_loop(0, 4096 // 16, zfill, 0)
    _prep_points(pts_hbm, cid, base_pt, fb, cb)

    tile_words = HALF // NTILES

    def compute_chunk(chk, hoff, idxb, valb):
        cbase = chk * (CH_GRP * 16)
        for g in range(CH_GRP):
            po = cbase + g * 16
            c0x = cb[0][pl.ds(po, 16)]
            c0y = cb[1][pl.ds(po, 16)]
            c0z = cb[2][pl.ds(po, 16)]
            e, cz0, cz1 = _corner_vectors(c0x, c0y, c0z)
            fx = fb[0][pl.ds(po, 16)]
            fy = fb[1][pl.ds(po, 16)]
            fz = fb[2][pl.ds(po, 16)]
            nv = nch[pl.ds(po, 16)]
            px1 = fx * nv
            px0 = nv - px1
            r = (px0 * (1.0 - fy), px0 * fy,
                 px1 * (1.0 - fy), px1 * fy)
            wz1 = fz
            wz0 = 1.0 - fz
            for cidx in range(4):
                for kz, czv, wzv in ((0, cz0, wz0), (1, cz1, wz1)):
                    loc = e[cidx] + czv - hoff
                    ok = (loc >= 0) & (loc < HALF)
                    idxb[g, pl.ds((cidx * 2 + kz) * 16, 16)] = (
                        jnp.where(ok, loc, DUMMY))
                    valb[g, pl.ds((cidx * 2 + kz) * 16, 16)] = r[cidx] * wzv
        return [pltpu.async_copy(valb.at[g], grid_sh.at[idxb.at[g]],
                                 sem, add=True)
                for g in range(CH_GRP)]

    for ch in range(3):
        pltpu.sync_copy(nrm_hbm.at[pl.ds((cid * 3 + ch) * NPAD + base_pt, PT)],
                        nch)

        def half_pass(h, _):
            hoff = h * HALF
            zcps = [pltpu.async_copy(
                zb, grid_sh.at[pl.ds(sid * tile_words + j * 4096, 4096)], sem2)
                for j in range(tile_words // 4096)]
            for cp in zcps:
                cp.wait()
            plsc.subcore_barrier()

            def chunk4(i, _):
                cps = []
                for sub in range(4):
                    cps += compute_chunk(i * 4 + sub, hoff, idxbs[sub],
                                         valbs[sub])
                for cp in cps:
                    cp.wait()
                return 0

            lax.fori_loop(0, NCHUNK // 4, chunk4, 0)
            plsc.subcore_barrier()
            ocps = [pltpu.async_copy(
                grid_sh.at[pl.ds(sid * tile_words + j * 4096, 4096)],
                ras_hbm.at[pl.ds((cid * 3 + ch) * G3 + hoff
                                 + sid * tile_words + j * 4096, 4096)],
                sem2)
                for j in range(tile_words // 4096)]
            for cp in ocps:
                cp.wait()
            plsc.subcore_barrier()
            return 0

        lax.fori_loop(0, 2, half_pass, 0)


def _dot(a, b):
    return jnp.dot(a, b, preferred_element_type=jnp.float32)


def _stage1_body(ras_ref, c_ref, s_ref, czs_ref, szs_ref, cys_ref, sys_ref,
                 br_ref, bi_ref):
    c = pl.program_id(1)
    a = ras_ref[0, 0, 0]
    mzc = jnp.where(c == 2, czs_ref[...], c_ref[...])
    mzs = jnp.where(c == 2, szs_ref[...], s_ref[...])
    zr = _dot(a, mzc)
    zi = -_dot(a, mzs)
    myc = jnp.where(c == 1, cys_ref[...], c_ref[...])
    mys = jnp.where(c == 1, sys_ref[...], s_ref[...])
    br_ref[0, 0, 0] = _dot(myc, zr) + _dot(mys, zi)
    bi_ref[0, 0, 0] = _dot(myc, zi) - _dot(mys, zr)


def _stage2_body(br_ref, bi_ref, c_ref, s_ref, cxs_ref, sxs_ref, g_ref,
                 fx2_ref, fyz2_ref, qr_ref, qi_ref):
    cm, sm = c_ref[...], s_ref[...]
    cxs, sxs = cxs_ref[...], sxs_ref[...]
    b0r, b0i = br_ref[0, 0], bi_ref[0, 0]
    b12r = br_ref[0, 1] + br_ref[0, 2]
    b12i = bi_ref[0, 1] + bi_ref[0, 2]
    dr = _dot(cxs, b0r) + _dot(sxs, b0i) + _dot(cm, b12r) + _dot(sm, b12i)
    di = _dot(cxs, b0i) - _dot(sxs, b0r) + _dot(cm, b12i) - _dot(sm, b12r)
    lap = fx2_ref[...] + fyz2_ref[...]
    m = g_ref[...] / (2.0 * np.pi * lap + EPS)
    pr = m * di
    pi = -(m * dr)
    shp = pr.shape
    row = lax.broadcasted_iota(jnp.int32, shp, 0)
    col = lax.broadcasted_iota(jnp.int32, shp, 1)
    dc = (row == 0) & (col == 0) & (pl.program_id(1) == 0)
    pr = jnp.where(dc, 0.0, pr)
    pi = jnp.where(dc, 0.0, pi)
    inv = np.float32(1.0 / N)
    qr_ref[0] = (_dot(cm, pr) - _dot(sm, pi)) * inv
    qi_ref[0] = (_dot(cm, pi) + _dot(sm, pr)) * inv


def _stage3_body(qr_ref, qi_ref, c_ref, s_ref, phi_ref):
    cm, sm = c_ref[...], s_ref[...]
    qr, qi = qr_ref[0, 0], qi_ref[0, 0]
    inv = np.float32(1.0 / N)
    rr = (_dot(cm, qr) - _dot(sm, qi)) * inv
    ri = (_dot(cm, qi) + _dot(sm, qr)) * inv
    phi_ref[0, 0] = (_dot(rr, cm) - _dot(ri, sm)) * inv


def _stage4_body(phi_ref, part_ref, out_ref):
    mean = jnp.sum(part_ref[...]) * np.float32(1.0 / NPTS)
    out_ref[0, 0] = phi_ref[0, 0] - mean


_MAT_SPEC = pl.BlockSpec((N, N), lambda *a: (0, 0))


def _run_tc_stages(ras, g2):
    f32 = jnp.float32
    br, bi = pl.pallas_call(
        _stage1_body,
        grid=(2, 3, N),
        in_specs=[pl.BlockSpec((1, 1, 1, N, N), lambda b, c, x: (b, c, x, 0, 0))]
        + [_MAT_SPEC] * 6,
        out_specs=[pl.BlockSpec((1, 1, 1, N, N), lambda b, c, x: (b, c, x, 0, 0))] * 2,
        out_shape=[jax.ShapeDtypeStruct((2, 3, N, N, N), f32)] * 2,
    )(ras, C_M, S_M, CZS, SZS, CYS, SYS)

    T = 2048
    NT = (N * N) // T
    br2 = br.reshape(2, 3, N, N * N)
    bi2 = bi.reshape(2, 3, N, N * N)
    qr, qi = pl.pallas_call(
        _stage2_body,
        grid=(2, NT),
        in_specs=[pl.BlockSpec((1, 3, N, T), lambda b, j: (b, 0, 0, j))] * 2
        + [_MAT_SPEC] * 4
        + [pl.BlockSpec((N, T), lambda b, j: (0, j)),
           pl.BlockSpec((N, 1), lambda b, j: (0, 0)),
           pl.BlockSpec((1, T), lambda b, j: (0, j))],
        out_specs=[pl.BlockSpec((1, N, T), lambda b, j: (b, 0, j))] * 2,
        out_shape=[jax.ShapeDtypeStruct((2, N, N * N), f32)] * 2,
    )(br2, bi2, C_M, S_M, CXS, SXS, g2, FX2, FYZ2)

    phi = pl.pallas_call(
        _stage3_body,
        grid=(2, N),
        in_specs=[pl.BlockSpec((1, 1, N, N), lambda b, x: (b, x, 0, 0))] * 2
        + [_MAT_SPEC] * 2,
        out_specs=pl.BlockSpec((1, 1, N, N), lambda b, x: (b, x, 0, 0)),
        out_shape=jax.ShapeDtypeStruct((2, N, N, N), f32),
    )(qr.reshape(2, N, N, N), qi.reshape(2, N, N, N), C_M, S_M)
    return phi


@functools.cache
def _sc_gather_kernel():
    return pl.kernel(
        _sc_gather_body,
        out_type=jax.ShapeDtypeStruct((2 * 16 * 16,), jnp.float32),
        mesh=_sc_mesh(),
        scratch_types=[
            [pltpu.VMEM((PT,), jnp.float32) for _ in range(3)],
            [pltpu.VMEM((PT,), jnp.int32) for _ in range(3)],
            pltpu.VMEM((CH_GRP, 128), jnp.int32),
            pltpu.VMEM((CH_GRP, 128), jnp.float32),
            pltpu.VMEM((CH_GRP, 128), jnp.float32),
            pltpu.VMEM((16,), jnp.float32),
            pltpu.SemaphoreType.DMA,
        ],
    )


def _sc_gather_body(pts_hbm, phi_hbm, out_hbm, fb, cb, idxb, wb, gatb, accb,
                    sem):
    cid = lax.axis_index("c")
    sid = lax.axis_index("s")
    base_pt = sid * PT
    _prep_points(pts_hbm, cid, base_pt, fb, cb)
    phi_off = cid * G3
    lane = lax.iota(jnp.int32, 16)

    def chunk(chk, acc):
        cbase = chk * (CH_GRP * 16)
        for g in range(CH_GRP):
            po = cbase + g * 16
            c0x = cb[0][pl.ds(po, 16)]
            c0y = cb[1][pl.ds(po, 16)]
            c0z = cb[2][pl.ds(po, 16)]
            e, cz0, cz1 = _corner_vectors(c0x, c0y, c0z)
            fx = fb[0][pl.ds(po, 16)]
            fy = fb[1][pl.ds(po, 16)]
            fz = fb[2][pl.ds(po, 16)]
            gpos = base_pt + po + lane
            msk = jnp.where(gpos < NPTS, 1.0, 0.0).astype(jnp.float32)
            px1 = fx * msk
            px0 = msk - px1
            r = (px0 * (1.0 - fy), px0 * fy, px1 * (1.0 - fy), px1 * fy)
            wz1 = fz
            wz0 = 1.0 - fz
            for cidx in range(4):
                for kz, czv, wzv in ((0, cz0, wz0), (1, cz1, wz1)):
                    sl = pl.ds((cidx * 2 + kz) * 16, 16)
                    idxb[g, sl] = e[cidx] + czv + phi_off
                    wb[g, sl] = r[cidx] * wzv
        cps = [pltpu.async_copy(phi_hbm.at[idxb.at[g]], gatb.at[g], sem)
               for g in range(CH_GRP)]
        for cp in cps:
            cp.wait()
        for g in range(CH_GRP):
            for k in range(8):
                sl = pl.ds(k * 16, 16)
                acc = acc + wb[g, sl] * gatb[g, sl]
        return acc

    acc = lax.fori_loop(0, NCHUNK, chunk, jnp.zeros((16,), jnp.float32))
    accb[...] = acc
    pltpu.sync_copy(accb, out_hbm.at[pl.ds((cid * 16 + sid) * 16, 16)])


def kernel(points, normals, u, g):
    del u
    f32 = jnp.float32
    pts_t = jnp.swapaxes(points, 1, 2)
    nrm_t = jnp.swapaxes(normals, 1, 2)
    npad = NPAD - points.shape[1]
    pts_p = jnp.pad(pts_t, ((0, 0), (0, 0), (0, npad)),
                    constant_values=0.5).reshape(-1)
    nrm_p = jnp.pad(nrm_t, ((0, 0), (0, 0), (0, npad)),
                    constant_values=0.0).reshape(-1)

    ras = _sc_scatter_kernel()(pts_p, nrm_p)
    g2 = g.reshape(N, N * N).astype(f32)
    phi = _run_tc_stages(ras.reshape(2, 3, N, N, N), g2)
    partials = _sc_gather_kernel()(pts_p, phi.reshape(2 * G3))
    out = pl.pallas_call(
        _stage4_body,
        grid=(2, N),
        in_specs=[pl.BlockSpec((1, 1, N, N), lambda b, x: (b, x, 0, 0)),
                  pl.BlockSpec((1, 1, 256), lambda b, x: (b, 0, 0))],
        out_specs=pl.BlockSpec((1, 1, N, N), lambda b, x: (b, x, 0, 0)),
        out_shape=jax.ShapeDtypeStruct((2, N, N, N), f32),
    )(phi, partials.reshape(2, 1, 256))
    return out


# speedup vs baseline: 2.9024x; 2.9024x over previous
import functools

import numpy as np
import jax
import jax.numpy as jnp
from jax import lax
from jax.experimental import pallas as pl
from jax.experimental.pallas import tpu as pltpu
from jax.experimental.pallas import tpu_sc as plsc

N = 128
G3 = N * N * N
HALF = G3 // 2
DUMMY = HALF
DUMMY_WORDS = 4096
EPS = 1e-6
SIGMA = 2

NPTS = 100000
NTILES = 16
PT = 6272
NPAD = NTILES * PT
GRP = PT // 16
CH_GRP = 14
NCHUNK = GRP // CH_GRP

_k = np.arange(N)
_ang = 2.0 * np.pi * ((np.outer(_k, _k)) % N) / N
_C = np.cos(_ang)
_S = np.sin(_ang)
_f = np.fft.fftfreq(N, d=1.0 / N)

C_M = np.asarray(_C, dtype=np.float32)
S_M = np.asarray(_S, dtype=np.float32)
CZS = np.asarray(_C * _f[None, :], dtype=np.float32)
SZS = np.asarray(_S * _f[None, :], dtype=np.float32)
CYS = np.asarray(_C * _f[:, None], dtype=np.float32)
SYS = np.asarray(_S * _f[:, None], dtype=np.float32)
CXS = CYS
SXS = SYS
FX2 = np.asarray((_f ** 2)[:, None], dtype=np.float32)
FYZ2 = np.asarray(((_f ** 2)[:, None] + (_f ** 2)[None, :]).reshape(1, -1),
                  dtype=np.float32)

@functools.cache
def _sc_mesh():
    return plsc.VectorSubcoreMesh(core_axis_name="c", subcore_axis_name="s",
                                  num_cores=2, num_subcores=16)


def _corner_vectors(c0x, c0y, c0z):
    ax0 = c0x * (N * N)
    ax1 = ((c0x + 1) & (N - 1)) * (N * N)
    by0 = c0y * N
    by1 = ((c0y + 1) & (N - 1)) * N
    cz1 = (c0z + 1) & (N - 1)
    e = (ax0 + by0, ax0 + by1, ax1 + by0, ax1 + by1)
    return e, c0z, cz1


def _prep_points(pts_hbm, cid, base_pt, fb, cb):
    for ax in range(3):
        off = (cid * 3 + ax) * NPAD + base_pt
        pltpu.sync_copy(pts_hbm.at[pl.ds(off, PT)], fb[ax])

    def prep(i, _):
        po = i * 16
        for ax in range(3):
            t = fb[ax][pl.ds(po, 16)] * float(N)
            c0 = t.astype(jnp.int32)
            cb[ax][pl.ds(po, 16)] = c0
            fb[ax][pl.ds(po, 16)] = t - c0.astype(jnp.float32)
        return 0

    lax.fori_loop(0, GRP, prep, 0)


@functools.cache
def _sc_scatter_kernel():
    return pl.kernel(
        _sc_scatter_body,
        out_type=jax.ShapeDtypeStruct((2 * 3 * G3,), jnp.float32),
        mesh=_sc_mesh(),
        scratch_types=[
            pltpu.VMEM_SHARED((HALF + DUMMY_WORDS,), jnp.float32),
            [pltpu.VMEM((PT,), jnp.float32) for _ in range(3)],
            [pltpu.VMEM((PT,), jnp.int32) for _ in range(3)],
            pltpu.VMEM((PT,), jnp.float32),
            [pltpu.VMEM((CH_GRP, 128), jnp.int32) for _ in range(4)],
            [pltpu.VMEM((CH_GRP, 128), jnp.float32) for _ in range(4)],
            pltpu.VMEM((4096,), jnp.float32),
            pltpu.SemaphoreType.DMA,
            pltpu.SemaphoreType.DMA,
        ],
    )


def _sc_scatter_body(pts_hbm, nrm_hbm, ras_hbm, grid_sh, fb, cb, nch, idxbs,
                     valbs, zb, sem, sem2):
    cid = lax.axis_index("c")
    sid = lax.axis_index("s")
    base_pt = sid * PT
    z16 = jnp.zeros((16,), jnp.float32)

    def zfill(i, _):
        zb[pl.ds(i * 16, 16)] = z16
        return 0

    lax.fori_loop(0, 4096 // 16, zfill, 0)
    _prep_points(pts_hbm, cid, base_pt, fb, cb)

    tile_words = HALF // NTILES

    def compute_chunk(chk, hoff, idxb, valb):
        cbase = chk * (CH_GRP * 16)
        for g in range(CH_GRP):
            po = cbase + g * 16
            c0x = cb[0][pl.ds(po, 16)]
            c0y = cb[1][pl.ds(po, 16)]
            c0z = cb[2][pl.ds(po, 16)]
            e, cz0, cz1 = _corner_vectors(c0x, c0y, c0z)
            fx = fb[0][pl.ds(po, 16)]
            fy = fb[1][pl.ds(po, 16)]
            fz = fb[2][pl.ds(po, 16)]
            nv = nch[pl.ds(po, 16)]
            px1 = fx * nv
            px0 = nv - px1
            r = (px0 * (1.0 - fy), px0 * fy,
                 px1 * (1.0 - fy), px1 * fy)
            wz1 = fz
            wz0 = 1.0 - fz
            for cidx in range(4):
                for kz, czv, wzv in ((0, cz0, wz0), (1, cz1, wz1)):
                    loc = e[cidx] + czv - hoff
                    ok = (loc >= 0) & (loc < HALF)
                    idxb[g, pl.ds((cidx * 2 + kz) * 16, 16)] = (
                        jnp.where(ok, loc, DUMMY + (loc & (DUMMY_WORDS - 1))))
                    valb[g, pl.ds((cidx * 2 + kz) * 16, 16)] = r[cidx] * wzv
        return [pltpu.async_copy(valb.at[g], grid_sh.at[idxb.at[g]],
                                 sem, add=True)
                for g in range(CH_GRP)]

    for ch in range(3):
        pltpu.sync_copy(nrm_hbm.at[pl.ds((cid * 3 + ch) * NPAD + base_pt, PT)],
                        nch)

        def half_pass(h, _):
            hoff = h * HALF
            zcps = [pltpu.async_copy(
                zb, grid_sh.at[pl.ds(sid * tile_words + j * 4096, 4096)], sem2)
                for j in range(tile_words // 4096)]
            for cp in zcps:
                cp.wait()
            plsc.subcore_barrier()

            def chunk4(i, _):
                cps = []
                for sub in range(4):
                    cps += compute_chunk(i * 4 + sub, hoff, idxbs[sub],
                                         valbs[sub])
                for cp in cps:
                    cp.wait()
                return 0

            lax.fori_loop(0, NCHUNK // 4, chunk4, 0)
            plsc.subcore_barrier()
            ocps = [pltpu.async_copy(
                grid_sh.at[pl.ds(sid * tile_words + j * 4096, 4096)],
                ras_hbm.at[pl.ds((cid * 3 + ch) * G3 + hoff
                                 + sid * tile_words + j * 4096, 4096)],
                sem2)
                for j in range(tile_words // 4096)]
            for cp in ocps:
                cp.wait()
            plsc.subcore_barrier()
            return 0

        lax.fori_loop(0, 2, half_pass, 0)


def _dot(a, b):
    return jnp.dot(a, b, preferred_element_type=jnp.float32)


def _stage1_body(ras_ref, c_ref, s_ref, czs_ref, szs_ref, cys_ref, sys_ref,
                 br_ref, bi_ref):
    c = pl.program_id(1)
    a = ras_ref[0, 0, 0]
    mzc = jnp.where(c == 2, czs_ref[...], c_ref[...])
    mzs = jnp.where(c == 2, szs_ref[...], s_ref[...])
    zr = _dot(a, mzc)
    zi = -_dot(a, mzs)
    myc = jnp.where(c == 1, cys_ref[...], c_ref[...])
    mys = jnp.where(c == 1, sys_ref[...], s_ref[...])
    br_ref[0, 0, 0] = _dot(myc, zr) + _dot(mys, zi)
    bi_ref[0, 0, 0] = _dot(myc, zi) - _dot(mys, zr)


def _stage2_body(br_ref, bi_ref, c_ref, s_ref, cxs_ref, sxs_ref, g_ref,
                 fx2_ref, fyz2_ref, qr_ref, qi_ref):
    cm, sm = c_ref[...], s_ref[...]
    cxs, sxs = cxs_ref[...], sxs_ref[...]
    b0r, b0i = br_ref[0, 0], bi_ref[0, 0]
    b12r = br_ref[0, 1] + br_ref[0, 2]
    b12i = bi_ref[0, 1] + bi_ref[0, 2]
    dr = _dot(cxs, b0r) + _dot(sxs, b0i) + _dot(cm, b12r) + _dot(sm, b12i)
    di = _dot(cxs, b0i) - _dot(sxs, b0r) + _dot(cm, b12i) - _dot(sm, b12r)
    lap = fx2_ref[...] + fyz2_ref[...]
    m = g_ref[...] / (2.0 * np.pi * lap + EPS)
    pr = m * di
    pi = -(m * dr)
    shp = pr.shape
    row = lax.broadcasted_iota(jnp.int32, shp, 0)
    col = lax.broadcasted_iota(jnp.int32, shp, 1)
    dc = (row == 0) & (col == 0) & (pl.program_id(1) == 0)
    pr = jnp.where(dc, 0.0, pr)
    pi = jnp.where(dc, 0.0, pi)
    inv = np.float32(1.0 / N)
    qr_ref[0] = (_dot(cm, pr) - _dot(sm, pi)) * inv
    qi_ref[0] = (_dot(cm, pi) + _dot(sm, pr)) * inv


def _stage3_body(qr_ref, qi_ref, c_ref, s_ref, phi_ref):
    cm, sm = c_ref[...], s_ref[...]
    qr, qi = qr_ref[0, 0], qi_ref[0, 0]
    inv = np.float32(1.0 / N)
    rr = (_dot(cm, qr) - _dot(sm, qi)) * inv
    ri = (_dot(cm, qi) + _dot(sm, qr)) * inv
    phi_ref[0, 0] = (_dot(rr, cm) - _dot(ri, sm)) * inv


def _stage4_body(phi_ref, part_ref, out_ref):
    mean = jnp.sum(part_ref[...]) * np.float32(1.0 / NPTS)
    out_ref[0, 0] = phi_ref[0, 0] - mean


_MAT_SPEC = pl.BlockSpec((N, N), lambda *a: (0, 0))


def _run_tc_stages(ras, g2):
    f32 = jnp.float32
    br, bi = pl.pallas_call(
        _stage1_body,
        grid=(2, 3, N),
        in_specs=[pl.BlockSpec((1, 1, 1, N, N), lambda b, c, x: (b, c, x, 0, 0))]
        + [_MAT_SPEC] * 6,
        out_specs=[pl.BlockSpec((1, 1, 1, N, N), lambda b, c, x: (b, c, x, 0, 0))] * 2,
        out_shape=[jax.ShapeDtypeStruct((2, 3, N, N, N), f32)] * 2,
    )(ras, C_M, S_M, CZS, SZS, CYS, SYS)

    T = 2048
    NT = (N * N) // T
    br2 = br.reshape(2, 3, N, N * N)
    bi2 = bi.reshape(2, 3, N, N * N)
    qr, qi = pl.pallas_call(
        _stage2_body,
        grid=(2, NT),
        in_specs=[pl.BlockSpec((1, 3, N, T), lambda b, j: (b, 0, 0, j))] * 2
        + [_MAT_SPEC] * 4
        + [pl.BlockSpec((N, T), lambda b, j: (0, j)),
           pl.BlockSpec((N, 1), lambda b, j: (0, 0)),
           pl.BlockSpec((1, T), lambda b, j: (0, j))],
        out_specs=[pl.BlockSpec((1, N, T), lambda b, j: (b, 0, j))] * 2,
        out_shape=[jax.ShapeDtypeStruct((2, N, N * N), f32)] * 2,
    )(br2, bi2, C_M, S_M, CXS, SXS, g2, FX2, FYZ2)

    phi = pl.pallas_call(
        _stage3_body,
        grid=(2, N),
        in_specs=[pl.BlockSpec((1, 1, N, N), lambda b, x: (b, x, 0, 0))] * 2
        + [_MAT_SPEC] * 2,
        out_specs=pl.BlockSpec((1, 1, N, N), lambda b, x: (b, x, 0, 0)),
        out_shape=jax.ShapeDtypeStruct((2, N, N, N), f32),
    )(qr.reshape(2, N, N, N), qi.reshape(2, N, N, N), C_M, S_M)
    return phi


@functools.cache
def _sc_gather_kernel():
    return pl.kernel(
        _sc_gather_body,
        out_type=jax.ShapeDtypeStruct((2 * 16 * 16,), jnp.float32),
        mesh=_sc_mesh(),
        scratch_types=[
            [pltpu.VMEM((PT,), jnp.float32) for _ in range(3)],
            [pltpu.VMEM((PT,), jnp.int32) for _ in range(3)],
            pltpu.VMEM((CH_GRP, 128), jnp.int32),
            pltpu.VMEM((CH_GRP, 128), jnp.float32),
            pltpu.VMEM((CH_GRP, 128), jnp.float32),
            pltpu.VMEM((16,), jnp.float32),
            pltpu.SemaphoreType.DMA,
        ],
    )


def _sc_gather_body(pts_hbm, phi_hbm, out_hbm, fb, cb, idxb, wb, gatb, accb,
                    sem):
    cid = lax.axis_index("c")
    sid = lax.axis_index("s")
    base_pt = sid * PT
    _prep_points(pts_hbm, cid, base_pt, fb, cb)
    phi_off = cid * G3
    lane = lax.iota(jnp.int32, 16)

    def chunk(chk, acc):
        cbase = chk * (CH_GRP * 16)
        for g in range(CH_GRP):
            po = cbase + g * 16
            c0x = cb[0][pl.ds(po, 16)]
            c0y = cb[1][pl.ds(po, 16)]
            c0z = cb[2][pl.ds(po, 16)]
            e, cz0, cz1 = _corner_vectors(c0x, c0y, c0z)
            fx = fb[0][pl.ds(po, 16)]
            fy = fb[1][pl.ds(po, 16)]
            fz = fb[2][pl.ds(po, 16)]
            gpos = base_pt + po + lane
            msk = jnp.where(gpos < NPTS, 1.0, 0.0).astype(jnp.float32)
            px1 = fx * msk
            px0 = msk - px1
            r = (px0 * (1.0 - fy), px0 * fy, px1 * (1.0 - fy), px1 * fy)
            wz1 = fz
            wz0 = 1.0 - fz
            for cidx in range(4):
                for kz, czv, wzv in ((0, cz0, wz0), (1, cz1, wz1)):
                    sl = pl.ds((cidx * 2 + kz) * 16, 16)
                    idxb[g, sl] = e[cidx] + czv + phi_off
                    wb[g, sl] = r[cidx] * wzv
        cps = [pltpu.async_copy(phi_hbm.at[idxb.at[g]], gatb.at[g], sem)
               for g in range(CH_GRP)]
        for cp in cps:
            cp.wait()
        for g in range(CH_GRP):
            for k in range(8):
                sl = pl.ds(k * 16, 16)
                acc = acc + wb[g, sl] * gatb[g, sl]
        return acc

    acc = lax.fori_loop(0, NCHUNK, chunk, jnp.zeros((16,), jnp.float32))
    accb[...] = acc
    pltpu.sync_copy(accb, out_hbm.at[pl.ds((cid * 16 + sid) * 16, 16)])


def kernel(points, normals, u, g):
    del u
    f32 = jnp.float32
    pts_t = jnp.swapaxes(points, 1, 2)
    nrm_t = jnp.swapaxes(normals, 1, 2)
    npad = NPAD - points.shape[1]
    pts_p = jnp.pad(pts_t, ((0, 0), (0, 0), (0, npad)),
                    constant_values=0.5).reshape(-1)
    nrm_p = jnp.pad(nrm_t, ((0, 0), (0, 0), (0, npad)),
                    constant_values=0.0).reshape(-1)

    ras = _sc_scatter_kernel()(pts_p, nrm_p)
    g2 = g.reshape(N, N * N).astype(f32)
    phi = _run_tc_stages(ras.reshape(2, 3, N, N, N), g2)
    partials = _sc_gather_kernel()(pts_p, phi.reshape(2 * G3))
    out = pl.pallas_call(
        _stage4_body,
        grid=(2, N),
        in_specs=[pl.BlockSpec((1, 1, N, N), lambda b, x: (b, x, 0, 0)),
                  pl.BlockSpec((1, 1, 256), lambda b, x: (b, 0, 0))],
        out_specs=pl.BlockSpec((1, 1, N, N), lambda b, x: (b, x, 0, 0)),
        out_shape=jax.ShapeDtypeStruct((2, N, N, N), f32),
    )(phi, partials.reshape(2, 1, 256))
    return out


# speedup vs baseline: 3.7427x; 1.2895x over previous
import functools

import numpy as np
import jax
import jax.numpy as jnp
from jax import lax
from jax.experimental import pallas as pl
from jax.experimental.pallas import tpu as pltpu
from jax.experimental.pallas import tpu_sc as plsc

N = 128
G3 = N * N * N
HALF = G3 // 2
DUMMY = HALF
DUMMY_WORDS = 4096
EPS = 1e-6
SIGMA = 2

NPTS = 100000
NTILES = 16
PT = 6272
NPAD = NTILES * PT
GRP = PT // 16
CH_GRP = 14
NCHUNK = GRP // CH_GRP

_k = np.arange(N)
_ang = 2.0 * np.pi * ((np.outer(_k, _k)) % N) / N
_C = np.cos(_ang)
_S = np.sin(_ang)
_f = np.fft.fftfreq(N, d=1.0 / N)

C_M = np.asarray(_C, dtype=np.float32)
S_M = np.asarray(_S, dtype=np.float32)
CZS = np.asarray(_C * _f[None, :], dtype=np.float32)
SZS = np.asarray(_S * _f[None, :], dtype=np.float32)
CYS = np.asarray(_C * _f[:, None], dtype=np.float32)
SYS = np.asarray(_S * _f[:, None], dtype=np.float32)
CXS = CYS
SXS = SYS
FX2 = np.asarray((_f ** 2)[:, None], dtype=np.float32)
FYZ2 = np.asarray(((_f ** 2)[:, None] + (_f ** 2)[None, :]).reshape(1, -1),
                  dtype=np.float32)

@functools.cache
def _sc_mesh():
    return plsc.VectorSubcoreMesh(core_axis_name="c", subcore_axis_name="s",
                                  num_cores=2, num_subcores=16)


def _corner_vectors(c0x, c0y, c0z):
    ax0 = c0x * (N * N)
    ax1 = ((c0x + 1) & (N - 1)) * (N * N)
    by0 = c0y * N
    by1 = ((c0y + 1) & (N - 1)) * N
    cz1 = (c0z + 1) & (N - 1)
    e = (ax0 + by0, ax0 + by1, ax1 + by0, ax1 + by1)
    return e, c0z, cz1


def _prep_points(pts_hbm, cid, base_pt, fb, cb):
    for ax in range(3):
        off = (cid * 3 + ax) * NPAD + base_pt
        pltpu.sync_copy(pts_hbm.at[pl.ds(off, PT)], fb[ax])

    def prep(i, _):
        po = i * 16
        for ax in range(3):
            t = fb[ax][pl.ds(po, 16)] * float(N)
            c0 = t.astype(jnp.int32)
            cb[ax][pl.ds(po, 16)] = c0
            fb[ax][pl.ds(po, 16)] = t - c0.astype(jnp.float32)
        return 0

    lax.fori_loop(0, GRP, prep, 0)


@functools.cache
def _sc_scatter_kernel():
    return pl.kernel(
        _sc_scatter_body,
        out_type=jax.ShapeDtypeStruct((2 * 3 * G3,), jnp.float32),
        mesh=_sc_mesh(),
        scratch_types=[
            pltpu.VMEM_SHARED((HALF + DUMMY_WORDS,), jnp.float32),
            [pltpu.VMEM((PT,), jnp.float32) for _ in range(3)],
            [pltpu.VMEM((PT,), jnp.int32) for _ in range(3)],
            pltpu.VMEM((PT,), jnp.float32),
            [pltpu.VMEM((CH_GRP, 128), jnp.int32) for _ in range(4)],
            [pltpu.VMEM((CH_GRP, 128), jnp.float32) for _ in range(4)],
            pltpu.VMEM((4096,), jnp.float32),
            pltpu.SemaphoreType.DMA,
            pltpu.SemaphoreType.DMA,
        ],
    )


def _sc_scatter_body(pts_hbm, nrm_hbm, ras_hbm, grid_sh, fb, cb, nch, idxbs,
                     valbs, zb, sem, sem2):
    cid = lax.axis_index("c")
    sid = lax.axis_index("s")
    base_pt = sid * PT
    z16 = jnp.zeros((16,), jnp.float32)

    def zfill(i, _):
        zb[pl.ds(i * 16, 16)] = z16
        return 0

    lax.fori_loop(0, 4096 // 16, zfill, 0)
    _prep_points(pts_hbm, cid, base_pt, fb, cb)

    tile_words = HALF // NTILES

    def compute_chunk(chk, hoff, idxb, valb):
        cbase = chk * (CH_GRP * 16)
        for g in range(CH_GRP):
            po = cbase + g * 16
            c0x = cb[0][pl.ds(po, 16)]
            c0y = cb[1][pl.ds(po, 16)]
            c0z = cb[2][pl.ds(po, 16)]
            e, cz0, cz1 = _corner_vectors(c0x, c0y, c0z)
            fx = fb[0][pl.ds(po, 16)]
            fy = fb[1][pl.ds(po, 16)]
            fz = fb[2][pl.ds(po, 16)]
            nv = nch[pl.ds(po, 16)]
            px1 = fx * nv
            px0 = nv - px1
            r = (px0 * (1.0 - fy), px0 * fy,
                 px1 * (1.0 - fy), px1 * fy)
            wz1 = fz
            wz0 = 1.0 - fz
            for cidx in range(4):
                for kz, czv, wzv in ((0, cz0, wz0), (1, cz1, wz1)):
                    loc = e[cidx] + czv - hoff
                    ok = (loc >= 0) & (loc < HALF)
                    idxb[g, pl.ds((cidx * 2 + kz) * 16, 16)] = (
                        jnp.where(ok, loc, DUMMY + (loc & (DUMMY_WORDS - 1))))
                    valb[g, pl.ds((cidx * 2 + kz) * 16, 16)] = r[cidx] * wzv
        return [pltpu.async_copy(valb.at[g], grid_sh.at[idxb.at[g]],
                                 sem, add=True)
                for g in range(CH_GRP)]

    for ch in range(3):
        pltpu.sync_copy(nrm_hbm.at[pl.ds((cid * 3 + ch) * NPAD + base_pt, PT)],
                        nch)

        def half_pass(h, _):
            hoff = h * HALF
            zcps = [pltpu.async_copy(
                zb, grid_sh.at[pl.ds(sid * tile_words + j * 4096, 4096)], sem2)
                for j in range(tile_words // 4096)]
            for cp in zcps:
                cp.wait()
            plsc.subcore_barrier()

            def chunk4(i, _):
                cps = []
                for sub in range(4):
                    cps += compute_chunk(i * 4 + sub, hoff, idxbs[sub],
                                         valbs[sub])
                for cp in cps:
                    cp.wait()
                return 0

            lax.fori_loop(0, NCHUNK // 4, chunk4, 0)
            plsc.subcore_barrier()
            ocps = [pltpu.async_copy(
                grid_sh.at[pl.ds(sid * tile_words + j * 4096, 4096)],
                ras_hbm.at[pl.ds((cid * 3 + ch) * G3 + hoff
                                 + sid * tile_words + j * 4096, 4096)],
                sem2)
                for j in range(tile_words // 4096)]
            for cp in ocps:
                cp.wait()
            plsc.subcore_barrier()
            return 0

        lax.fori_loop(0, 2, half_pass, 0)


def _dot(a, b):
    return jnp.dot(a, b, preferred_element_type=jnp.float32)


def _stage1_body(ras_ref, c_ref, s_ref, czs_ref, szs_ref, cys_ref, sys_ref,
                 br_ref, bi_ref):
    cm, sm = c_ref[...], s_ref[...]
    a0 = ras_ref[0, 0, 0]
    zr = _dot(a0, cm)
    zi = -_dot(a0, sm)
    br_ref[0, 0, 0] = _dot(cm, zr) + _dot(sm, zi)
    bi_ref[0, 0, 0] = _dot(cm, zi) - _dot(sm, zr)
    a1 = ras_ref[0, 1, 0]
    zr = _dot(a1, cm)
    zi = -_dot(a1, sm)
    cys, sys_ = cys_ref[...], sys_ref[...]
    b1r = _dot(cys, zr) + _dot(sys_, zi)
    b1i = _dot(cys, zi) - _dot(sys_, zr)
    a2 = ras_ref[0, 2, 0]
    zr = _dot(a2, czs_ref[...])
    zi = -_dot(a2, szs_ref[...])
    br_ref[0, 1, 0] = b1r + _dot(cm, zr) + _dot(sm, zi)
    bi_ref[0, 1, 0] = b1i + _dot(cm, zi) - _dot(sm, zr)


def _stage2_body(br_ref, bi_ref, c_ref, s_ref, cxs_ref, sxs_ref, g_ref,
                 fx2_ref, fyz2_ref, qr_ref, qi_ref):
    cm, sm = c_ref[...], s_ref[...]
    cxs, sxs = cxs_ref[...], sxs_ref[...]
    b0r, b0i = br_ref[0, 0], bi_ref[0, 0]
    b12r = br_ref[0, 1]
    b12i = bi_ref[0, 1]
    dr = _dot(cxs, b0r) + _dot(sxs, b0i) + _dot(cm, b12r) + _dot(sm, b12i)
    di = _dot(cxs, b0i) - _dot(sxs, b0r) + _dot(cm, b12i) - _dot(sm, b12r)
    lap = fx2_ref[...] + fyz2_ref[...]
    m = g_ref[...] / (2.0 * np.pi * lap + EPS)
    pr = m * di
    pi = -(m * dr)
    shp = pr.shape
    row = lax.broadcasted_iota(jnp.int32, shp, 0)
    col = lax.broadcasted_iota(jnp.int32, shp, 1)
    dc = (row == 0) & (col == 0) & (pl.program_id(1) == 0)
    pr = jnp.where(dc, 0.0, pr)
    pi = jnp.where(dc, 0.0, pi)
    inv = np.float32(1.0 / N)
    qr_ref[0] = (_dot(cm, pr) - _dot(sm, pi)) * inv
    qi_ref[0] = (_dot(cm, pi) + _dot(sm, pr)) * inv


def _stage3_body(qr_ref, qi_ref, c_ref, s_ref, phi_ref):
    cm, sm = c_ref[...], s_ref[...]
    qr, qi = qr_ref[0, 0], qi_ref[0, 0]
    inv = np.float32(1.0 / N)
    rr = (_dot(cm, qr) - _dot(sm, qi)) * inv
    ri = (_dot(cm, qi) + _dot(sm, qr)) * inv
    phi_ref[0, 0] = (_dot(rr, cm) - _dot(ri, sm)) * inv


def _stage4_body(phi_ref, part_ref, out_ref):
    mean = jnp.sum(part_ref[...]) * np.float32(1.0 / NPTS)
    out_ref[0, 0] = phi_ref[0, 0] - mean


_MAT_SPEC = pl.BlockSpec((N, N), lambda *a: (0, 0))


def _run_tc_stages(ras, g2):
    f32 = jnp.float32
    br, bi = pl.pallas_call(
        _stage1_body,
        grid=(2, N),
        in_specs=[pl.BlockSpec((1, 3, 1, N, N), lambda b, x: (b, 0, x, 0, 0))]
        + [_MAT_SPEC] * 6,
        out_specs=[pl.BlockSpec((1, 2, 1, N, N), lambda b, x: (b, 0, x, 0, 0))] * 2,
        out_shape=[jax.ShapeDtypeStruct((2, 2, N, N, N), f32)] * 2,
    )(ras, C_M, S_M, CZS, SZS, CYS, SYS)

    T = 2048
    NT = (N * N) // T
    br2 = br.reshape(2, 2, N, N * N)
    bi2 = bi.reshape(2, 2, N, N * N)
    qr, qi = pl.pallas_call(
        _stage2_body,
        grid=(2, NT),
        in_specs=[pl.BlockSpec((1, 2, N, T), lambda b, j: (b, 0, 0, j))] * 2
        + [_MAT_SPEC] * 4
        + [pl.BlockSpec((N, T), lambda b, j: (0, j)),
           pl.BlockSpec((N, 1), lambda b, j: (0, 0)),
           pl.BlockSpec((1, T), lambda b, j: (0, j))],
        out_specs=[pl.BlockSpec((1, N, T), lambda b, j: (b, 0, j))] * 2,
        out_shape=[jax.ShapeDtypeStruct((2, N, N * N), f32)] * 2,
    )(br2, bi2, C_M, S_M, CXS, SXS, g2, FX2, FYZ2)

    phi = pl.pallas_call(
        _stage3_body,
        grid=(2, N),
        in_specs=[pl.BlockSpec((1, 1, N, N), lambda b, x: (b, x, 0, 0))] * 2
        + [_MAT_SPEC] * 2,
        out_specs=pl.BlockSpec((1, 1, N, N), lambda b, x: (b, x, 0, 0)),
        out_shape=jax.ShapeDtypeStruct((2, N, N, N), f32),
    )(qr.reshape(2, N, N, N), qi.reshape(2, N, N, N), C_M, S_M)
    return phi


@functools.cache
def _sc_gather_kernel():
    return pl.kernel(
        _sc_gather_body,
        out_type=jax.ShapeDtypeStruct((2 * 16 * 16,), jnp.float32),
        mesh=_sc_mesh(),
        scratch_types=[
            [pltpu.VMEM((PT,), jnp.float32) for _ in range(3)],
            [pltpu.VMEM((PT,), jnp.int32) for _ in range(3)],
            pltpu.VMEM((CH_GRP, 128), jnp.int32),
            pltpu.VMEM((CH_GRP, 128), jnp.float32),
            pltpu.VMEM((CH_GRP, 128), jnp.float32),
            pltpu.VMEM((16,), jnp.float32),
            pltpu.SemaphoreType.DMA,
        ],
    )


def _sc_gather_body(pts_hbm, phi_hbm, out_hbm, fb, cb, idxb, wb, gatb, accb,
                    sem):
    cid = lax.axis_index("c")
    sid = lax.axis_index("s")
    base_pt = sid * PT
    _prep_points(pts_hbm, cid, base_pt, fb, cb)
    phi_off = cid * G3
    lane = lax.iota(jnp.int32, 16)

    def chunk(chk, acc):
        cbase = chk * (CH_GRP * 16)
        for g in range(CH_GRP):
            po = cbase + g * 16
            c0x = cb[0][pl.ds(po, 16)]
            c0y = cb[1][pl.ds(po, 16)]
            c0z = cb[2][pl.ds(po, 16)]
            e, cz0, cz1 = _corner_vectors(c0x, c0y, c0z)
            fx = fb[0][pl.ds(po, 16)]
            fy = fb[1][pl.ds(po, 16)]
            fz = fb[2][pl.ds(po, 16)]
            gpos = base_pt + po + lane
            msk = jnp.where(gpos < NPTS, 1.0, 0.0).astype(jnp.float32)
            px1 = fx * msk
            px0 = msk - px1
            r = (px0 * (1.0 - fy), px0 * fy, px1 * (1.0 - fy), px1 * fy)
            wz1 = fz
            wz0 = 1.0 - fz
            for cidx in range(4):
                for kz, czv, wzv in ((0, cz0, wz0), (1, cz1, wz1)):
                    sl = pl.ds((cidx * 2 + kz) * 16, 16)
                    idxb[g, sl] = e[cidx] + czv + phi_off
                    wb[g, sl] = r[cidx] * wzv
        cps = [pltpu.async_copy(phi_hbm.at[idxb.at[g]], gatb.at[g], sem)
               for g in range(CH_GRP)]
        for cp in cps:
            cp.wait()
        for g in range(CH_GRP):
            for k in range(8):
                sl = pl.ds(k * 16, 16)
                acc = acc + wb[g, sl] * gatb[g, sl]
        return acc

    acc = lax.fori_loop(0, NCHUNK, chunk, jnp.zeros((16,), jnp.float32))
    accb[...] = acc
    pltpu.sync_copy(accb, out_hbm.at[pl.ds((cid * 16 + sid) * 16, 16)])


def kernel(points, normals, u, g):
    del u
    f32 = jnp.float32
    pts_t = jnp.swapaxes(points, 1, 2)
    nrm_t = jnp.swapaxes(normals, 1, 2)
    npad = NPAD - points.shape[1]
    pts_p = jnp.pad(pts_t, ((0, 0), (0, 0), (0, npad)),
                    constant_values=0.5).reshape(-1)
    nrm_p = jnp.pad(nrm_t, ((0, 0), (0, 0), (0, npad)),
                    constant_values=0.0).reshape(-1)

    ras = _sc_scatter_kernel()(pts_p, nrm_p)
    g2 = g.reshape(N, N * N).astype(f32)
    phi = _run_tc_stages(ras.reshape(2, 3, N, N, N), g2)
    partials = _sc_gather_kernel()(pts_p, phi.reshape(2 * G3))
    out = pl.pallas_call(
        _stage4_body,
        grid=(2, N),
        in_specs=[pl.BlockSpec((1, 1, N, N), lambda b, x: (b, x, 0, 0)),
                  pl.BlockSpec((1, 1, 256), lambda b, x: (b, 0, 0))],
        out_specs=pl.BlockSpec((1, 1, N, N), lambda b, x: (b, x, 0, 0)),
        out_shape=jax.ShapeDtypeStruct((2, N, N, N), f32),
    )(phi, partials.reshape(2, 1, 256))
    return out


# speedup vs baseline: 4.9533x; 1.3234x over previous
import functools

import numpy as np
import jax
import jax.numpy as jnp
from jax import lax
from jax.experimental import pallas as pl
from jax.experimental.pallas import tpu as pltpu
from jax.experimental.pallas import tpu_sc as plsc

N = 128
G3 = N * N * N
HALF = G3 // 2
DUMMY = HALF
DUMMY_WORDS = 4096
EPS = 1e-6
SIGMA = 2

NPTS = 100000
NTILES = 16
PT = 6272
NPAD = NTILES * PT
GRP = PT // 16
CH_GRP = 14
NCHUNK = GRP // CH_GRP

_k = np.arange(N)
_ang = 2.0 * np.pi * ((np.outer(_k, _k)) % N) / N
_C = np.cos(_ang)
_S = np.sin(_ang)
_f = np.fft.fftfreq(N, d=1.0 / N)

C_M = np.asarray(_C, dtype=np.float32)
S_M = np.asarray(_S, dtype=np.float32)
CZS = np.asarray(_C * _f[None, :], dtype=np.float32)
SZS = np.asarray(_S * _f[None, :], dtype=np.float32)
CYS = np.asarray(_C * _f[:, None], dtype=np.float32)
SYS = np.asarray(_S * _f[:, None], dtype=np.float32)
CXS = CYS
SXS = SYS
FX2 = np.asarray((_f ** 2)[:, None], dtype=np.float32)
FYZ2 = np.asarray(((_f ** 2)[:, None] + (_f ** 2)[None, :]).reshape(1, -1),
                  dtype=np.float32)

@functools.cache
def _sc_mesh():
    return plsc.VectorSubcoreMesh(core_axis_name="c", subcore_axis_name="s",
                                  num_cores=2, num_subcores=16)


def _corner_vectors(c0x, c0y, c0z):
    ax0 = c0x * (N * N)
    ax1 = ((c0x + 1) & (N - 1)) * (N * N)
    by0 = c0y * N
    by1 = ((c0y + 1) & (N - 1)) * N
    cz1 = (c0z + 1) & (N - 1)
    e = (ax0 + by0, ax0 + by1, ax1 + by0, ax1 + by1)
    return e, c0z, cz1


def _prep_points(pts_hbm, cid, base_pt, fb, cb):
    for ax in range(3):
        off = (cid * 3 + ax) * NPAD + base_pt
        pltpu.sync_copy(pts_hbm.at[pl.ds(off, PT)], fb[ax])

    def prep(i, _):
        po = i * 16
        for ax in range(3):
            t = fb[ax][pl.ds(po, 16)] * float(N)
            c0 = t.astype(jnp.int32)
            cb[ax][pl.ds(po, 16)] = c0
            fb[ax][pl.ds(po, 16)] = t - c0.astype(jnp.float32)
        return 0

    lax.fori_loop(0, GRP, prep, 0)


@functools.cache
def _sc_scatter_kernel():
    return pl.kernel(
        _sc_scatter_body,
        out_type=jax.ShapeDtypeStruct((2 * 3 * G3,), jnp.float32),
        mesh=_sc_mesh(),
        scratch_types=[
            pltpu.VMEM_SHARED((HALF + DUMMY_WORDS,), jnp.float32),
            [pltpu.VMEM((PT,), jnp.float32) for _ in range(3)],
            [pltpu.VMEM((PT,), jnp.int32) for _ in range(3)],
            pltpu.VMEM((PT,), jnp.float32),
            [pltpu.VMEM((CH_GRP, 128), jnp.int32) for _ in range(4)],
            [pltpu.VMEM((CH_GRP, 128), jnp.float32) for _ in range(4)],
            pltpu.VMEM((4096,), jnp.float32),
            pltpu.SemaphoreType.DMA,
            pltpu.SemaphoreType.DMA,
        ],
    )


def _sc_scatter_body(pts_hbm, nrm_hbm, ras_hbm, grid_sh, fb, cb, nch, idxbs,
                     valbs, zb, sem, sem2):
    cid = lax.axis_index("c")
    sid = lax.axis_index("s")
    base_pt = sid * PT
    z16 = jnp.zeros((16,), jnp.float32)

    def zfill(i, _):
        zb[pl.ds(i * 16, 16)] = z16
        return 0

    lax.fori_loop(0, 4096 // 16, zfill, 0)
    _prep_points(pts_hbm, cid, base_pt, fb, cb)

    tile_words = HALF // NTILES

    def compute_chunk(chk, hoff, idxb, valb):
        cbase = chk * (CH_GRP * 16)
        for g in range(CH_GRP):
            po = cbase + g * 16
            c0x = cb[0][pl.ds(po, 16)]
            c0y = cb[1][pl.ds(po, 16)]
            c0z = cb[2][pl.ds(po, 16)]
            e, cz0, cz1 = _corner_vectors(c0x, c0y, c0z)
            fx = fb[0][pl.ds(po, 16)]
            fy = fb[1][pl.ds(po, 16)]
            fz = fb[2][pl.ds(po, 16)]
            nv = nch[pl.ds(po, 16)]
            px1 = fx * nv
            px0 = nv - px1
            r = (px0 * (1.0 - fy), px0 * fy,
                 px1 * (1.0 - fy), px1 * fy)
            wz1 = fz
            wz0 = 1.0 - fz
            for cidx in range(4):
                for kz, czv, wzv in ((0, cz0, wz0), (1, cz1, wz1)):
                    loc = e[cidx] + czv - hoff
                    ok = (loc >= 0) & (loc < HALF)
                    idxb[g, pl.ds((cidx * 2 + kz) * 16, 16)] = (
                        jnp.where(ok, loc, DUMMY + (loc & (DUMMY_WORDS - 1))))
                    valb[g, pl.ds((cidx * 2 + kz) * 16, 16)] = r[cidx] * wzv
        return [pltpu.async_copy(valb.at[g], grid_sh.at[idxb.at[g]],
                                 sem, add=True)
                for g in range(CH_GRP)]

    for ch in range(3):
        pltpu.sync_copy(nrm_hbm.at[pl.ds((cid * 3 + ch) * NPAD + base_pt, PT)],
                        nch)

        def half_pass(h, _):
            hoff = h * HALF
            zcps = [pltpu.async_copy(
                zb, grid_sh.at[pl.ds(sid * tile_words + j * 4096, 4096)], sem2)
                for j in range(tile_words // 4096)]
            for cp in zcps:
                cp.wait()
            plsc.subcore_barrier()

            def chunk4(i, _):
                cps = []
                for sub in range(4):
                    cps += compute_chunk(i * 4 + sub, hoff, idxbs[sub],
                                         valbs[sub])
                for cp in cps:
                    cp.wait()
                return 0

            lax.fori_loop(0, NCHUNK // 4, chunk4, 0)
            plsc.subcore_barrier()
            ocps = [pltpu.async_copy(
                grid_sh.at[pl.ds(sid * tile_words + j * 4096, 4096)],
                ras_hbm.at[pl.ds((cid * 3 + ch) * G3 + hoff
                                 + sid * tile_words + j * 4096, 4096)],
                sem2)
                for j in range(tile_words // 4096)]
            for cp in ocps:
                cp.wait()
            plsc.subcore_barrier()
            return 0

        lax.fori_loop(0, 2, half_pass, 0)


def _dot(a, b):
    return jnp.dot(a, b, preferred_element_type=jnp.float32)


def _stage1_body(ras_ref, c_ref, s_ref, czs_ref, szs_ref, cys_ref, sys_ref,
                 br_ref, bi_ref):
    cm, sm = c_ref[...], s_ref[...]
    cys, sys_ = cys_ref[...], sys_ref[...]
    for sl in range(2):
        a0 = ras_ref[0, 0, sl]
        zr = _dot(a0, cm)
        zi = -_dot(a0, sm)
        br_ref[0, 0, sl] = _dot(cm, zr) + _dot(sm, zi)
        bi_ref[0, 0, sl] = _dot(cm, zi) - _dot(sm, zr)
        a1 = ras_ref[0, 1, sl]
        zr = _dot(a1, cm)
        zi = -_dot(a1, sm)
        b1r = _dot(cys, zr) + _dot(sys_, zi)
        b1i = _dot(cys, zi) - _dot(sys_, zr)
        a2 = ras_ref[0, 2, sl]
        zr = _dot(a2, czs_ref[...])
        zi = -_dot(a2, szs_ref[...])
        br_ref[0, 1, sl] = b1r + _dot(cm, zr) + _dot(sm, zi)
        bi_ref[0, 1, sl] = b1i + _dot(cm, zi) - _dot(sm, zr)


def _stage2_body(br_ref, bi_ref, c_ref, s_ref, cxs_ref, sxs_ref, g_ref,
                 fx2_ref, fyz2_ref, qr_ref, qi_ref):
    cm, sm = c_ref[...], s_ref[...]
    cxs, sxs = cxs_ref[...], sxs_ref[...]
    b0r, b0i = br_ref[0, 0], bi_ref[0, 0]
    b12r = br_ref[0, 1]
    b12i = bi_ref[0, 1]
    dr = _dot(cxs, b0r) + _dot(sxs, b0i) + _dot(cm, b12r) + _dot(sm, b12i)
    di = _dot(cxs, b0i) - _dot(sxs, b0r) + _dot(cm, b12i) - _dot(sm, b12r)
    lap = fx2_ref[...] + fyz2_ref[...]
    m = g_ref[...] / (2.0 * np.pi * lap + EPS)
    pr = m * di
    pi = -(m * dr)
    shp = pr.shape
    row = lax.broadcasted_iota(jnp.int32, shp, 0)
    col = lax.broadcasted_iota(jnp.int32, shp, 1)
    dc = (row == 0) & (col == 0) & (pl.program_id(1) == 0)
    pr = jnp.where(dc, 0.0, pr)
    pi = jnp.where(dc, 0.0, pi)
    inv = np.float32(1.0 / N)
    qr_ref[0] = (_dot(cm, pr) - _dot(sm, pi)) * inv
    qi_ref[0] = (_dot(cm, pi) + _dot(sm, pr)) * inv


def _stage3_body(qr_ref, qi_ref, c_ref, s_ref, phi_ref):
    cm, sm = c_ref[...], s_ref[...]
    inv = np.float32(1.0 / N)
    for sl in range(4):
        qr, qi = qr_ref[0, sl], qi_ref[0, sl]
        rr = (_dot(cm, qr) - _dot(sm, qi)) * inv
        ri = (_dot(cm, qi) + _dot(sm, qr)) * inv
        phi_ref[0, sl] = (_dot(rr, cm) - _dot(ri, sm)) * inv


def _stage4_body(phi_ref, part_ref, out_ref):
    mean = jnp.sum(part_ref[...]) * np.float32(1.0 / NPTS)
    out_ref[0] = phi_ref[0] - mean


_MAT_SPEC = pl.BlockSpec((N, N), lambda *a: (0, 0))


def _run_tc_stages(ras, g2):
    f32 = jnp.float32
    br, bi = pl.pallas_call(
        _stage1_body,
        grid=(2, N // 2),
        in_specs=[pl.BlockSpec((1, 3, 2, N, N), lambda b, x: (b, 0, x, 0, 0))]
        + [_MAT_SPEC] * 6,
        out_specs=[pl.BlockSpec((1, 2, 2, N, N), lambda b, x: (b, 0, x, 0, 0))] * 2,
        out_shape=[jax.ShapeDtypeStruct((2, 2, N, N, N), f32)] * 2,
    )(ras, C_M, S_M, CZS, SZS, CYS, SYS)

    T = 2048
    NT = (N * N) // T
    br2 = br.reshape(2, 2, N, N * N)
    bi2 = bi.reshape(2, 2, N, N * N)
    qr, qi = pl.pallas_call(
        _stage2_body,
        grid=(2, NT),
        in_specs=[pl.BlockSpec((1, 2, N, T), lambda b, j: (b, 0, 0, j))] * 2
        + [_MAT_SPEC] * 4
        + [pl.BlockSpec((N, T), lambda b, j: (0, j)),
           pl.BlockSpec((N, 1), lambda b, j: (0, 0)),
           pl.BlockSpec((1, T), lambda b, j: (0, j))],
        out_specs=[pl.BlockSpec((1, N, T), lambda b, j: (b, 0, j))] * 2,
        out_shape=[jax.ShapeDtypeStruct((2, N, N * N), f32)] * 2,
    )(br2, bi2, C_M, S_M, CXS, SXS, g2, FX2, FYZ2)

    phi = pl.pallas_call(
        _stage3_body,
        grid=(2, N // 4),
        in_specs=[pl.BlockSpec((1, 4, N, N), lambda b, x: (b, x, 0, 0))] * 2
        + [_MAT_SPEC] * 2,
        out_specs=pl.BlockSpec((1, 4, N, N), lambda b, x: (b, x, 0, 0)),
        out_shape=jax.ShapeDtypeStruct((2, N, N, N), f32),
    )(qr.reshape(2, N, N, N), qi.reshape(2, N, N, N), C_M, S_M)
    return phi


@functools.cache
def _sc_gather_kernel():
    return pl.kernel(
        _sc_gather_body,
        out_type=jax.ShapeDtypeStruct((2 * 16 * 16,), jnp.float32),
        mesh=_sc_mesh(),
        scratch_types=[
            [pltpu.VMEM((PT,), jnp.float32) for _ in range(3)],
            [pltpu.VMEM((PT,), jnp.int32) for _ in range(3)],
            pltpu.VMEM((CH_GRP, 128), jnp.int32),
            pltpu.VMEM((CH_GRP, 128), jnp.float32),
            pltpu.VMEM((CH_GRP, 128), jnp.float32),
            pltpu.VMEM((16,), jnp.float32),
            pltpu.SemaphoreType.DMA,
        ],
    )


def _sc_gather_body(pts_hbm, phi_hbm, out_hbm, fb, cb, idxb, wb, gatb, accb,
                    sem):
    cid = lax.axis_index("c")
    sid = lax.axis_index("s")
    base_pt = sid * PT
    _prep_points(pts_hbm, cid, base_pt, fb, cb)
    phi_off = cid * G3
    lane = lax.iota(jnp.int32, 16)

    def chunk(chk, acc):
        cbase = chk * (CH_GRP * 16)
        for g in range(CH_GRP):
            po = cbase + g * 16
            c0x = cb[0][pl.ds(po, 16)]
            c0y = cb[1][pl.ds(po, 16)]
            c0z = cb[2][pl.ds(po, 16)]
            e, cz0, cz1 = _corner_vectors(c0x, c0y, c0z)
            fx = fb[0][pl.ds(po, 16)]
            fy = fb[1][pl.ds(po, 16)]
            fz = fb[2][pl.ds(po, 16)]
            gpos = base_pt + po + lane
            msk = jnp.where(gpos < NPTS, 1.0, 0.0).astype(jnp.float32)
            px1 = fx * msk
            px0 = msk - px1
            r = (px0 * (1.0 - fy), px0 * fy, px1 * (1.0 - fy), px1 * fy)
            wz1 = fz
            wz0 = 1.0 - fz
            for cidx in range(4):
                for kz, czv, wzv in ((0, cz0, wz0), (1, cz1, wz1)):
                    sl = pl.ds((cidx * 2 + kz) * 16, 16)
                    idxb[g, sl] = e[cidx] + czv + phi_off
                    wb[g, sl] = r[cidx] * wzv
        cps = [pltpu.async_copy(phi_hbm.at[idxb.at[g]], gatb.at[g], sem)
               for g in range(CH_GRP)]
        for cp in cps:
            cp.wait()
        for g in range(CH_GRP):
            for k in range(8):
                sl = pl.ds(k * 16, 16)
                acc = acc + wb[g, sl] * gatb[g, sl]
        return acc

    acc = lax.fori_loop(0, NCHUNK, chunk, jnp.zeros((16,), jnp.float32))
    accb[...] = acc
    pltpu.sync_copy(accb, out_hbm.at[pl.ds((cid * 16 + sid) * 16, 16)])


def kernel(points, normals, u, g):
    del u
    f32 = jnp.float32
    pts_t = jnp.swapaxes(points, 1, 2)
    nrm_t = jnp.swapaxes(normals, 1, 2)
    npad = NPAD - points.shape[1]
    pts_p = jnp.pad(pts_t, ((0, 0), (0, 0), (0, npad)),
                    constant_values=0.5).reshape(-1)
    nrm_p = jnp.pad(nrm_t, ((0, 0), (0, 0), (0, npad)),
                    constant_values=0.0).reshape(-1)

    ras = _sc_scatter_kernel()(pts_p, nrm_p)
    g2 = g.reshape(N, N * N).astype(f32)
    phi = _run_tc_stages(ras.reshape(2, 3, N, N, N), g2)
    partials = _sc_gather_kernel()(pts_p, phi.reshape(2 * G3))
    out = pl.pallas_call(
        _stage4_body,
        grid=(2, N // 4),
        in_specs=[pl.BlockSpec((1, 4, N, N), lambda b, x: (b, x, 0, 0)),
                  pl.BlockSpec((1, 1, 256), lambda b, x: (b, 0, 0))],
        out_specs=pl.BlockSpec((1, 4, N, N), lambda b, x: (b, x, 0, 0)),
        out_shape=jax.ShapeDtypeStruct((2, N, N, N), f32),
    )(phi, partials.reshape(2, 1, 256))
    return out


# speedup vs baseline: 5.0344x; 1.0164x over previous
import functools

import numpy as np
import jax
import jax.numpy as jnp
from jax import lax
from jax.experimental import pallas as pl
from jax.experimental.pallas import tpu as pltpu
from jax.experimental.pallas import tpu_sc as plsc

N = 128
G3 = N * N * N
HALF = G3 // 2
DUMMY = HALF
DUMMY_WORDS = 4096
EPS = 1e-6
SIGMA = 2

NPTS = 100000
NTILES = 16
PT = 6272
NPAD = NTILES * PT
GRP = PT // 16
CH_GRP = 14
NCHUNK = GRP // CH_GRP

_k = np.arange(N)
_ang = 2.0 * np.pi * ((np.outer(_k, _k)) % N) / N
_C = np.cos(_ang)
_S = np.sin(_ang)
_f = np.fft.fftfreq(N, d=1.0 / N)

C_M = np.asarray(_C, dtype=np.float32)
S_M = np.asarray(_S, dtype=np.float32)
CZS = np.asarray(_C * _f[None, :], dtype=np.float32)
SZS = np.asarray(_S * _f[None, :], dtype=np.float32)
CYS = np.asarray(_C * _f[:, None], dtype=np.float32)
SYS = np.asarray(_S * _f[:, None], dtype=np.float32)
CXS = CYS
SXS = SYS
FX2 = np.asarray((_f ** 2)[:, None], dtype=np.float32)
FYZ2 = np.asarray(((_f ** 2)[:, None] + (_f ** 2)[None, :]).reshape(1, -1),
                  dtype=np.float32)

@functools.cache
def _sc_mesh():
    return plsc.VectorSubcoreMesh(core_axis_name="c", subcore_axis_name="s",
                                  num_cores=2, num_subcores=16)


def _corner_vectors(c0x, c0y, c0z):
    ax0 = c0x * (N * N)
    ax1 = ((c0x + 1) & (N - 1)) * (N * N)
    by0 = c0y * N
    by1 = ((c0y + 1) & (N - 1)) * N
    cz1 = (c0z + 1) & (N - 1)
    e = (ax0 + by0, ax0 + by1, ax1 + by0, ax1 + by1)
    return e, c0z, cz1


def _prep_points(pts_hbm, cid, base_pt, fb, cb):
    for ax in range(3):
        off = (cid * 3 + ax) * NPAD + base_pt
        pltpu.sync_copy(pts_hbm.at[pl.ds(off, PT)], fb[ax])

    def prep(i, _):
        po = i * 16
        for ax in range(3):
            t = fb[ax][pl.ds(po, 16)] * float(N)
            c0 = t.astype(jnp.int32)
            cb[ax][pl.ds(po, 16)] = c0
            fb[ax][pl.ds(po, 16)] = t - c0.astype(jnp.float32)
        return 0

    lax.fori_loop(0, GRP, prep, 0)


@functools.cache
def _sc_scatter_kernel():
    return pl.kernel(
        _sc_scatter_body,
        out_type=jax.ShapeDtypeStruct((2 * 3 * G3,), jnp.float32),
        mesh=_sc_mesh(),
        scratch_types=[
            pltpu.VMEM_SHARED((HALF + DUMMY_WORDS,), jnp.float32),
            [pltpu.VMEM((PT,), jnp.float32) for _ in range(3)],
            [pltpu.VMEM((PT,), jnp.int32) for _ in range(3)],
            pltpu.VMEM((PT,), jnp.float32),
            [pltpu.VMEM((CH_GRP, 128), jnp.int32) for _ in range(4)],
            [pltpu.VMEM((CH_GRP, 128), jnp.float32) for _ in range(4)],
            pltpu.VMEM((4096,), jnp.float32),
            pltpu.SemaphoreType.DMA,
            pltpu.SemaphoreType.DMA,
        ],
    )


def _sc_scatter_body(pts_hbm, nrm_hbm, ras_hbm, grid_sh, fb, cb, nch, idxbs,
                     valbs, zb, sem, sem2):
    cid = lax.axis_index("c")
    sid = lax.axis_index("s")
    base_pt = sid * PT
    z16 = jnp.zeros((16,), jnp.float32)

    def zfill(i, _):
        zb[pl.ds(i * 16, 16)] = z16
        return 0

    lax.fori_loop(0, 4096 // 16, zfill, 0)
    _prep_points(pts_hbm, cid, base_pt, fb, cb)

    tile_words = HALF // NTILES

    def compute_chunk(chk, hoff, idxb, valb):
        cbase = chk * (CH_GRP * 16)
        for g in range(CH_GRP):
            po = cbase + g * 16
            c0x = cb[0][pl.ds(po, 16)]
            c0y = cb[1][pl.ds(po, 16)]
            c0z = cb[2][pl.ds(po, 16)]
            e, cz0, cz1 = _corner_vectors(c0x, c0y, c0z)
            fx = fb[0][pl.ds(po, 16)]
            fy = fb[1][pl.ds(po, 16)]
            fz = fb[2][pl.ds(po, 16)]
            nv = nch[pl.ds(po, 16)]
            px1 = fx * nv
            px0 = nv - px1
            r = (px0 * (1.0 - fy), px0 * fy,
                 px1 * (1.0 - fy), px1 * fy)
            wz1 = fz
            wz0 = 1.0 - fz
            for cidx in range(4):
                for kz, czv, wzv in ((0, cz0, wz0), (1, cz1, wz1)):
                    loc = e[cidx] + czv - hoff
                    ok = (loc >= 0) & (loc < HALF)
                    idxb[g, pl.ds((cidx * 2 + kz) * 16, 16)] = (
                        jnp.where(ok, loc, DUMMY + (loc & (DUMMY_WORDS - 1))))
                    valb[g, pl.ds((cidx * 2 + kz) * 16, 16)] = r[cidx] * wzv
        return [pltpu.async_copy(valb.at[g], grid_sh.at[idxb.at[g]],
                                 sem, add=True)
                for g in range(CH_GRP)]

    for ch in range(3):
        pltpu.sync_copy(nrm_hbm.at[pl.ds((cid * 3 + ch) * NPAD + base_pt, PT)],
                        nch)

        def half_pass(h, _):
            hoff = h * HALF
            zcps = [pltpu.async_copy(
                zb, grid_sh.at[pl.ds(sid * tile_words + j * 4096, 4096)], sem2)
                for j in range(tile_words // 4096)]
            for cp in zcps:
                cp.wait()
            plsc.subcore_barrier()

            def chunk4(i, _):
                cps = []
                for sub in range(4):
                    cps += compute_chunk(i * 4 + sub, hoff, idxbs[sub],
                                         valbs[sub])
                for cp in cps:
                    cp.wait()
                return 0

            lax.fori_loop(0, NCHUNK // 4, chunk4, 0)
            plsc.subcore_barrier()
            ocps = [pltpu.async_copy(
                grid_sh.at[pl.ds(sid * tile_words + j * 4096, 4096)],
                ras_hbm.at[pl.ds((cid * 3 + ch) * G3 + hoff
                                 + sid * tile_words + j * 4096, 4096)],
                sem2)
                for j in range(tile_words // 4096)]
            for cp in ocps:
                cp.wait()
            plsc.subcore_barrier()
            return 0

        lax.fori_loop(0, 2, half_pass, 0)


def _dot(a, b):
    return jnp.dot(a, b, preferred_element_type=jnp.float32)


def _stage1_body(ras_ref, c_ref, s_ref, czs_ref, szs_ref, cys_ref, sys_ref,
                 br_ref, bi_ref):
    cm, sm = c_ref[...], s_ref[...]
    cys, sys_ = cys_ref[...], sys_ref[...]
    for sl in range(4):
        a0 = ras_ref[0, 0, sl]
        zr = _dot(a0, cm)
        zi = -_dot(a0, sm)
        br_ref[0, 0, sl] = _dot(cm, zr) + _dot(sm, zi)
        bi_ref[0, 0, sl] = _dot(cm, zi) - _dot(sm, zr)
        a1 = ras_ref[0, 1, sl]
        zr = _dot(a1, cm)
        zi = -_dot(a1, sm)
        b1r = _dot(cys, zr) + _dot(sys_, zi)
        b1i = _dot(cys, zi) - _dot(sys_, zr)
        a2 = ras_ref[0, 2, sl]
        zr = _dot(a2, czs_ref[...])
        zi = -_dot(a2, szs_ref[...])
        br_ref[0, 1, sl] = b1r + _dot(cm, zr) + _dot(sm, zi)
        bi_ref[0, 1, sl] = b1i + _dot(cm, zi) - _dot(sm, zr)


def _stage2_body(br_ref, bi_ref, c_ref, s_ref, cxs_ref, sxs_ref, g_ref,
                 fx2_ref, fyz2_ref, qr_ref, qi_ref):
    cm, sm = c_ref[...], s_ref[...]
    cxs, sxs = cxs_ref[...], sxs_ref[...]
    b0r, b0i = br_ref[0, 0], bi_ref[0, 0]
    b12r = br_ref[0, 1]
    b12i = bi_ref[0, 1]
    dr = _dot(cxs, b0r) + _dot(sxs, b0i) + _dot(cm, b12r) + _dot(sm, b12i)
    di = _dot(cxs, b0i) - _dot(sxs, b0r) + _dot(cm, b12i) - _dot(sm, b12r)
    lap = fx2_ref[...] + fyz2_ref[...]
    m = g_ref[...] / (2.0 * np.pi * lap + EPS)
    pr = m * di
    pi = -(m * dr)
    shp = pr.shape
    row = lax.broadcasted_iota(jnp.int32, shp, 0)
    col = lax.broadcasted_iota(jnp.int32, shp, 1)
    dc = (row == 0) & (col == 0) & (pl.program_id(1) == 0)
    pr = jnp.where(dc, 0.0, pr)
    pi = jnp.where(dc, 0.0, pi)
    inv = np.float32(1.0 / N)
    qr_ref[0] = (_dot(cm, pr) - _dot(sm, pi)) * inv
    qi_ref[0] = (_dot(cm, pi) + _dot(sm, pr)) * inv


def _stage3_body(qr_ref, qi_ref, c_ref, s_ref, phi_ref):
    cm, sm = c_ref[...], s_ref[...]
    inv = np.float32(1.0 / N)
    for sl in range(4):
        qr, qi = qr_ref[0, sl], qi_ref[0, sl]
        rr = (_dot(cm, qr) - _dot(sm, qi)) * inv
        ri = (_dot(cm, qi) + _dot(sm, qr)) * inv
        phi_ref[0, sl] = (_dot(rr, cm) - _dot(ri, sm)) * inv


def _stage4_body(phi_ref, part_ref, out_ref):
    mean = jnp.sum(part_ref[...]) * np.float32(1.0 / NPTS)
    out_ref[0] = phi_ref[0] - mean


_MAT_SPEC = pl.BlockSpec((N, N), lambda *a: (0, 0))


def _run_tc_stages(ras, g2):
    f32 = jnp.float32
    br, bi = pl.pallas_call(
        _stage1_body,
        grid=(2, N // 4),
        in_specs=[pl.BlockSpec((1, 3, 4, N, N), lambda b, x: (b, 0, x, 0, 0))]
        + [_MAT_SPEC] * 6,
        out_specs=[pl.BlockSpec((1, 2, 4, N, N), lambda b, x: (b, 0, x, 0, 0))] * 2,
        out_shape=[jax.ShapeDtypeStruct((2, 2, N, N, N), f32)] * 2,
    )(ras, C_M, S_M, CZS, SZS, CYS, SYS)

    T = 2048
    NT = (N * N) // T
    br2 = br.reshape(2, 2, N, N * N)
    bi2 = bi.reshape(2, 2, N, N * N)
    qr, qi = pl.pallas_call(
        _stage2_body,
        grid=(2, NT),
        in_specs=[pl.BlockSpec((1, 2, N, T), lambda b, j: (b, 0, 0, j))] * 2
        + [_MAT_SPEC] * 4
        + [pl.BlockSpec((N, T), lambda b, j: (0, j)),
           pl.BlockSpec((N, 1), lambda b, j: (0, 0)),
           pl.BlockSpec((1, T), lambda b, j: (0, j))],
        out_specs=[pl.BlockSpec((1, N, T), lambda b, j: (b, 0, j))] * 2,
        out_shape=[jax.ShapeDtypeStruct((2, N, N * N), f32)] * 2,
    )(br2, bi2, C_M, S_M, CXS, SXS, g2, FX2, FYZ2)

    phi = pl.pallas_call(
        _stage3_body,
        grid=(2, N // 4),
        in_specs=[pl.BlockSpec((1, 4, N, N), lambda b, x: (b, x, 0, 0))] * 2
        + [_MAT_SPEC] * 2,
        out_specs=pl.BlockSpec((1, 4, N, N), lambda b, x: (b, x, 0, 0)),
        out_shape=jax.ShapeDtypeStruct((2, N, N, N), f32),
    )(qr.reshape(2, N, N, N), qi.reshape(2, N, N, N), C_M, S_M)
    return phi


@functools.cache
def _sc_gather_kernel():
    return pl.kernel(
        _sc_gather_body,
        out_type=jax.ShapeDtypeStruct((2 * 16 * 16,), jnp.float32),
        mesh=_sc_mesh(),
        scratch_types=[
            [pltpu.VMEM((PT,), jnp.float32) for _ in range(3)],
            [pltpu.VMEM((PT,), jnp.int32) for _ in range(3)],
            pltpu.VMEM((CH_GRP, 128), jnp.int32),
            pltpu.VMEM((CH_GRP, 128), jnp.float32),
            pltpu.VMEM((CH_GRP, 128), jnp.float32),
            pltpu.VMEM((16,), jnp.float32),
            pltpu.SemaphoreType.DMA,
        ],
    )


def _sc_gather_body(pts_hbm, phi_hbm, out_hbm, fb, cb, idxb, wb, gatb, accb,
                    sem):
    cid = lax.axis_index("c")
    sid = lax.axis_index("s")
    base_pt = sid * PT
    _prep_points(pts_hbm, cid, base_pt, fb, cb)
    phi_off = cid * G3
    lane = lax.iota(jnp.int32, 16)

    def chunk(chk, acc):
        cbase = chk * (CH_GRP * 16)
        for g in range(CH_GRP):
            po = cbase + g * 16
            c0x = cb[0][pl.ds(po, 16)]
            c0y = cb[1][pl.ds(po, 16)]
            c0z = cb[2][pl.ds(po, 16)]
            e, cz0, cz1 = _corner_vectors(c0x, c0y, c0z)
            fx = fb[0][pl.ds(po, 16)]
            fy = fb[1][pl.ds(po, 16)]
            fz = fb[2][pl.ds(po, 16)]
            gpos = base_pt + po + lane
            msk = jnp.where(gpos < NPTS, 1.0, 0.0).astype(jnp.float32)
            px1 = fx * msk
            px0 = msk - px1
            r = (px0 * (1.0 - fy), px0 * fy, px1 * (1.0 - fy), px1 * fy)
            wz1 = fz
            wz0 = 1.0 - fz
            for cidx in range(4):
                for kz, czv, wzv in ((0, cz0, wz0), (1, cz1, wz1)):
                    sl = pl.ds((cidx * 2 + kz) * 16, 16)
                    idxb[g, sl] = e[cidx] + czv + phi_off
                    wb[g, sl] = r[cidx] * wzv
        cps = [pltpu.async_copy(phi_hbm.at[idxb.at[g]], gatb.at[g], sem)
               for g in range(CH_GRP)]
        for cp in cps:
            cp.wait()
        for g in range(CH_GRP):
            for k in range(8):
                sl = pl.ds(k * 16, 16)
                acc = acc + wb[g, sl] * gatb[g, sl]
        return acc

    acc = lax.fori_loop(0, NCHUNK, chunk, jnp.zeros((16,), jnp.float32))
    accb[...] = acc
    pltpu.sync_copy(accb, out_hbm.at[pl.ds((cid * 16 + sid) * 16, 16)])


def kernel(points, normals, u, g):
    del u
    f32 = jnp.float32
    pts_t = jnp.swapaxes(points, 1, 2)
    nrm_t = jnp.swapaxes(normals, 1, 2)
    npad = NPAD - points.shape[1]
    pts_p = jnp.pad(pts_t, ((0, 0), (0, 0), (0, npad)),
                    constant_values=0.5).reshape(-1)
    nrm_p = jnp.pad(nrm_t, ((0, 0), (0, 0), (0, npad)),
                    constant_values=0.0).reshape(-1)

    ras = _sc_scatter_kernel()(pts_p, nrm_p)
    g2 = g.reshape(N, N * N).astype(f32)
    phi = _run_tc_stages(ras.reshape(2, 3, N, N, N), g2)
    partials = _sc_gather_kernel()(pts_p, phi.reshape(2 * G3))
    out = pl.pallas_call(
        _stage4_body,
        grid=(2, N // 4),
        in_specs=[pl.BlockSpec((1, 4, N, N), lambda b, x: (b, x, 0, 0)),
                  pl.BlockSpec((1, 1, 256), lambda b, x: (b, 0, 0))],
        out_specs=pl.BlockSpec((1, 4, N, N), lambda b, x: (b, x, 0, 0)),
        out_shape=jax.ShapeDtypeStruct((2, N, N, N), f32),
    )(phi, partials.reshape(2, 1, 256))
    return out
